# baseline (speedup 1.0000x reference)
import functools

import jax
import jax.numpy as jnp
from jax import lax
from jax.experimental import pallas as pl
from jax.experimental.pallas import tpu as pltpu
from jax.experimental.pallas import tpu_sc as plsc

NC, NS = 2, 16
NW = NC * NS
CHUNK = 128

_N = 10000
_E = 320000
_S = 5
_D = 128
_H = 64

_AGGROWS = 10240
_AGGTILE = _AGGROWS // NS
_PERW = 10240
_NCH = _PERW // CHUNK
_GCH = 40

_mesh = plsc.VectorSubcoreMesh(
    core_axis_name="c", subcore_axis_name="s", num_cores=NC, num_subcores=NS)


@functools.partial(
    pl.kernel,
    out_type=jax.ShapeDtypeStruct((NC * _S * _AGGROWS, _D), jnp.float32),
    mesh=_mesh,
    scratch_types=[
        pltpu.VMEM_SHARED((_AGGROWS, _D), jnp.float32),
        pltpu.VMEM((_NCH, CHUNK), jnp.int32),
        pltpu.VMEM((CHUNK, _D), jnp.float32),
        pltpu.SemaphoreType.DMA,
    ],
)
def _sc_deg(dsts_hbm, ones_hbm, zeros_hbm, out_hbm, deg_sp, dib, onesbuf, ssem):
    cid = lax.axis_index("c")
    sid = lax.axis_index("s")
    wid = sid * NC + cid
    nch = _NCH
    pltpu.sync_copy(ones_hbm, onesbuf)
    for s in range(_S):
        pltpu.sync_copy(zeros_hbm, deg_sp.at[pl.ds(sid * _AGGTILE, _AGGTILE)])
        rowbase = (s * NW + wid) * _NCH
        pltpu.sync_copy(dsts_hbm.at[pl.ds(rowbase, _NCH)], dib)
        plsc.subcore_barrier()

        def fire(j, c):
            pltpu.async_copy(onesbuf, deg_sp.at[dib.at[j]], ssem, add=True)
            return c

        lax.fori_loop(0, nch, fire, 0)

        def drain(j, c):
            pltpu.make_async_copy(onesbuf, deg_sp.at[dib.at[j]], ssem).wait()
            return c

        lax.fori_loop(0, nch, drain, 0)
        plsc.subcore_barrier()
        pltpu.sync_copy(
            deg_sp.at[pl.ds(sid * _AGGTILE, _AGGTILE)],
            out_hbm.at[pl.ds((cid * _S + s) * _AGGROWS + sid * _AGGTILE,
                             _AGGTILE)])


@functools.partial(
    pl.kernel,
    out_type=jax.ShapeDtypeStruct((NC * _S * _AGGROWS, _D), jnp.float32),
    mesh=_mesh,
    scratch_types=[
        pltpu.VMEM_SHARED((_AGGROWS, _D), jnp.float32),
        pltpu.VMEM((_GCH, CHUNK), jnp.int32),
        pltpu.VMEM((_GCH, CHUNK), jnp.int32),
        pltpu.VMEM((CHUNK, _D), jnp.float32),
        pltpu.VMEM((CHUNK, _D), jnp.float32),
        pltpu.SemaphoreType.DMA,
        pltpu.SemaphoreType.DMA,
        pltpu.SemaphoreType.DMA,
        pltpu.SemaphoreType.DMA,
    ],
)
def _sc_agg(srcs_hbm, dsts_hbm, hp_hbm, zeros_hbm, out_hbm,
            agg_sp, sib, dib, rows0, rows1, gs0, gs1, ss0, ss1):
    cid = lax.axis_index("c")
    sid = lax.axis_index("s")
    wid = sid * NC + cid
    rows = (rows0, rows1)
    gsem = (gs0, gs1)
    ssem = (ss0, ss1)

    def gstart(j, b):
        pltpu.async_copy(hp_hbm.at[sib.at[j]], rows[b], gsem[b])

    def gwait(j, b):
        pltpu.make_async_copy(hp_hbm.at[sib.at[j]], rows[b], gsem[b]).wait()

    def sstart(j, b):
        pltpu.async_copy(rows[b], agg_sp.at[dib.at[j]], ssem[b], add=True)

    def swait(j, b):
        pltpu.make_async_copy(rows[b], agg_sp.at[dib.at[j]], ssem[b]).wait()

    for s in range(_S):
        pltpu.sync_copy(zeros_hbm, agg_sp.at[pl.ds(sid * _AGGTILE, _AGGTILE)])
        plsc.subcore_barrier()
        for g in range(_NCH // _GCH):
            rowbase = (s * NW + wid) * _NCH + g * _GCH
            pltpu.sync_copy(srcs_hbm.at[pl.ds(rowbase, _GCH)], sib)
            pltpu.sync_copy(dsts_hbm.at[pl.ds(rowbase, _GCH)], dib)

            def win(kk, c):
                j0 = kk * 2
                gstart(j0, 0)
                gstart(j0 + 1, 1)
                gwait(j0, 0)
                gwait(j0 + 1, 1)
                sstart(j0, 0)
                sstart(j0 + 1, 1)
                swait(j0, 0)
                swait(j0 + 1, 1)
                return c

            lax.fori_loop(0, _GCH // 2, win, 0)
        plsc.subcore_barrier()
        pltpu.sync_copy(
            agg_sp.at[pl.ds(sid * _AGGTILE, _AGGTILE)],
            out_hbm.at[pl.ds((cid * _S + s) * _AGGROWS + sid * _AGGTILE,
                             _AGGTILE)])


_R = 400


def _dis_from_deg(deg_blk):
    deg = deg_blk[0, :, :, 0] + deg_blk[1, :, :, 0] + 1.0
    return lax.rsqrt(deg)


def _k_pre_body(x_ref, w1_ref, deg_ref, p_ref, hp1_ref):
    x = x_ref[...]
    p = jnp.dot(x, w1_ref[...], preferred_element_type=jnp.float32,
                precision=lax.Precision.HIGHEST)
    p_ref[...] = p
    dis = _dis_from_deg(deg_ref[...])
    hp1_ref[...] = dis[:, :, None] * p[None, :, :]


def _k_mid_body(agg_ref, hp1_ref, x_ref, b1_ref, deg_ref, w2_ref,
                x1_ref, hp2_ref):
    dis = _dis_from_deg(deg_ref[...])
    aggsum = agg_ref[0] + agg_ref[1]
    conv = dis[:, :, None] * (aggsum + hp1_ref[...]) + b1_ref[...][None]
    x1 = jax.nn.relu(conv) + x_ref[...][None]
    x1_ref[...] = x1
    q = jnp.dot(x1, w2_ref[...], preferred_element_type=jnp.float32,
                precision=lax.Precision.HIGHEST)
    hp2_ref[...] = dis[:, :, None] * q


def _k_post_body(agg_ref, hp2_ref, x1_ref, b2_ref, deg_ref, mask_ref,
                 wc1_ref, bc1_ref, wc2_ref, bc2_ref, out_ref):
    dis = _dis_from_deg(deg_ref[...])
    x2 = (dis[:, :, None] * (agg_ref[0] + agg_ref[1] + hp2_ref[...])
          + b2_ref[...][None] + x1_ref[...])
    a = jnp.mean(x2, axis=0) * mask_ref[...]
    h = jax.nn.relu(jnp.dot(a, wc1_ref[...], preferred_element_type=jnp.float32,
                            precision=lax.Precision.HIGHEST) + bc1_ref[...])
    logit = jnp.dot(h, wc2_ref[...], preferred_element_type=jnp.float32,
                    precision=lax.Precision.HIGHEST) + bc2_ref[...]
    out_ref[...] = jax.nn.sigmoid(logit)


def _full(shape):
    return pl.BlockSpec(shape, lambda i: (0,) * len(shape))


def _rows(shape, axis):
    def imap(i, axis=axis, rank=len(shape)):
        return tuple(i if d == axis else 0 for d in range(rank))
    return pl.BlockSpec(shape, imap)


def kernel(node_features, edge_index, post_mask, W1, b1, W2, b2,
           Wc1, bc1, Wc2, bc2):
    N, D = node_features.shape
    S = edge_index.shape[0]
    E = edge_index.shape[2]
    H = Wc1.shape[1]

    shift = (jnp.arange(S, dtype=jnp.int32) * N)[:, None]

    def _staged(idx, padval):
        a = idx.reshape(S, NW, E // NW)
        a = jnp.pad(a, ((0, 0), (0, 0), (0, _PERW - E // NW)),
                    constant_values=padval)
        return a.reshape(S * NW * _NCH, CHUNK)

    srcs_flat = _staged(edge_index[:, 0, :] + shift, 0)
    dsts_flat = _staged(edge_index[:, 1, :], _AGGROWS - 1)

    ones_rows = jnp.zeros((CHUNK, D), jnp.float32).at[:, 0].set(1.0)
    zeros_agg = jnp.zeros((_AGGTILE, D), jnp.float32)
    maskf = post_mask.astype(jnp.float32).reshape(N, 1)
    b1r = b1.reshape(1, D)
    b2r = b2.reshape(1, D)
    bc1r = bc1.reshape(1, H)

    degtab = _sc_deg(dsts_flat, ones_rows, zeros_agg)
    deg4 = degtab.reshape(NC, S, _AGGROWS, D)[:, :, :N, :1]

    nb = N // _R
    p, hp1 = pl.pallas_call(
        _k_pre_body,
        grid=(nb,),
        in_specs=[_rows((_R, D), 0), _full((D, D)),
                  _rows((NC, S, _R, 1), 2)],
        out_specs=[_rows((_R, D), 0), _rows((S, _R, D), 1)],
        out_shape=[jax.ShapeDtypeStruct((N, D), jnp.float32),
                   jax.ShapeDtypeStruct((S, N, D), jnp.float32)],
    )(node_features, W1, deg4)

    agg1 = _sc_agg(srcs_flat, dsts_flat, hp1.reshape(S * N, D), zeros_agg)
    agg1 = agg1.reshape(NC, S, _AGGROWS, D)[:, :, :N]

    x1, hp2 = pl.pallas_call(
        _k_mid_body,
        grid=(nb,),
        in_specs=[_rows((NC, S, _R, D), 2), _rows((S, _R, D), 1),
                  _rows((_R, D), 0), _full((1, D)),
                  _rows((NC, S, _R, 1), 2), _full((D, D))],
        out_specs=[_rows((S, _R, D), 1), _rows((S, _R, D), 1)],
        out_shape=[jax.ShapeDtypeStruct((S, N, D), jnp.float32),
                   jax.ShapeDtypeStruct((S, N, D), jnp.float32)],
    )(agg1, hp1, node_features, b1r, deg4, W2)

    agg2 = _sc_agg(srcs_flat, dsts_flat, hp2.reshape(S * N, D), zeros_agg)
    agg2 = agg2.reshape(NC, S, _AGGROWS, D)[:, :, :N]

    out = pl.pallas_call(
        _k_post_body,
        grid=(nb,),
        in_specs=[_rows((NC, S, _R, D), 2), _rows((S, _R, D), 1),
                  _rows((S, _R, D), 1), _full((1, D)),
                  _rows((NC, S, _R, 1), 2), _rows((_R, 1), 0),
                  _full((D, H)), _full((1, H)), _full((H, 1)), _full((1, 1))],
        out_specs=[_rows((_R, 1), 0)],
        out_shape=[jax.ShapeDtypeStruct((N, 1), jnp.float32)],
    )(agg2, hp2, x1, b2r, deg4, maskf, Wc1, bc1r, Wc2, bc2.reshape(1, 1))[0]
    return out.reshape(N)


# speedup vs baseline: 1.8454x; 1.8454x over previous
import functools

import jax
import jax.numpy as jnp
from jax import lax
from jax.experimental import pallas as pl
from jax.experimental.pallas import tpu as pltpu
from jax.experimental.pallas import tpu_sc as plsc

NC, NS = 2, 16
NW = NC * NS
CHUNK = 80

_N = 10000
_E = 320000
_S = 5
_D = 128
_H = 64

_AGGROWS = 10240
_AGGTILE = _AGGROWS // NS
_PERW = 10000
_NCH = _PERW // CHUNK
_NCHP = 128

_mesh = plsc.VectorSubcoreMesh(
    core_axis_name="c", subcore_axis_name="s", num_cores=NC, num_subcores=NS)


@functools.partial(
    pl.kernel,
    out_type=jax.ShapeDtypeStruct((NC * _S * _AGGROWS, _D), jnp.float32),
    mesh=_mesh,
    scratch_types=[
        pltpu.VMEM_SHARED((_AGGROWS, _D), jnp.float32),
        pltpu.VMEM((_NCHP, CHUNK), jnp.int32),
        pltpu.VMEM((CHUNK, _D), jnp.float32),
        pltpu.SemaphoreType.DMA,
    ],
)
def _sc_deg(dsts_hbm, ones_hbm, zeros_hbm, out_hbm, deg_sp, dib, onesbuf, ssem):
    cid = lax.axis_index("c")
    sid = lax.axis_index("s")
    wid = sid * NC + cid
    nch = _NCH
    pltpu.sync_copy(ones_hbm, onesbuf)
    for s in range(_S):
        pltpu.sync_copy(zeros_hbm, deg_sp.at[pl.ds(sid * _AGGTILE, _AGGTILE)])
        rowbase = (s * NW + wid) * _NCHP
        pltpu.sync_copy(dsts_hbm.at[pl.ds(rowbase, _NCHP)], dib)
        plsc.subcore_barrier()

        def fire(j, c):
            pltpu.async_copy(onesbuf, deg_sp.at[dib.at[j]], ssem, add=True)
            return c

        lax.fori_loop(0, nch, fire, 0)

        def drain(j, c):
            pltpu.make_async_copy(onesbuf, deg_sp.at[dib.at[j]], ssem).wait()
            return c

        lax.fori_loop(0, nch, drain, 0)
        plsc.subcore_barrier()
        pltpu.sync_copy(
            deg_sp.at[pl.ds(sid * _AGGTILE, _AGGTILE)],
            out_hbm.at[pl.ds((cid * _S + s) * _AGGROWS + sid * _AGGTILE,
                             _AGGTILE)])


@functools.partial(
    pl.kernel,
    out_type=jax.ShapeDtypeStruct((NC * _S * _AGGROWS, _D), jnp.float32),
    mesh=_mesh,
    scratch_types=[
        pltpu.VMEM_SHARED((_AGGROWS, _D), jnp.float32),
        pltpu.VMEM((_NCHP, CHUNK), jnp.int32),
        pltpu.VMEM((_NCHP, CHUNK), jnp.int32),
        pltpu.VMEM((CHUNK, _D), jnp.float32),
        pltpu.SemaphoreType.DMA,
    ],
)
def _sc_agg(srcs_hbm, dsts_hbm, hp_hbm, zeros_hbm, out_hbm,
            agg_sp, sib, dib, rows, sem):
    cid = lax.axis_index("c")
    sid = lax.axis_index("s")
    wid = sid * NC + cid
    for s in range(_S):
        pltpu.sync_copy(zeros_hbm, agg_sp.at[pl.ds(sid * _AGGTILE, _AGGTILE)])
        rowbase = (s * NW + wid) * _NCHP
        pltpu.sync_copy(srcs_hbm.at[pl.ds(rowbase, _NCHP)], sib)
        pltpu.sync_copy(dsts_hbm.at[pl.ds(rowbase, _NCHP)], dib)
        plsc.subcore_barrier()

        def body(j, c):
            pltpu.async_copy(hp_hbm.at[sib.at[j]], rows, sem).wait()
            pltpu.sync_copy(rows, agg_sp.at[dib.at[j]], add=True)
            return c

        lax.fori_loop(0, _NCH, body, 0)
        plsc.subcore_barrier()
        pltpu.sync_copy(
            agg_sp.at[pl.ds(sid * _AGGTILE, _AGGTILE)],
            out_hbm.at[pl.ds((cid * _S + s) * _AGGROWS + sid * _AGGTILE,
                             _AGGTILE)])


_R = 400


def _dis_from_deg(deg_blk):
    deg = deg_blk[0, :, :, 0] + deg_blk[1, :, :, 0] + 1.0
    return lax.rsqrt(deg)


def _k_pre_body(x_ref, w1_ref, deg_ref, p_ref, hp1_ref):
    x = x_ref[...]
    p = jnp.dot(x, w1_ref[...], preferred_element_type=jnp.float32,
                precision=lax.Precision.HIGHEST)
    p_ref[...] = p
    dis = _dis_from_deg(deg_ref[...])
    hp1_ref[...] = dis[:, :, None] * p[None, :, :]


def _k_mid_body(agg_ref, hp1_ref, x_ref, b1_ref, deg_ref, w2_ref,
                x1_ref, hp2_ref):
    dis = _dis_from_deg(deg_ref[...])
    aggsum = agg_ref[0] + agg_ref[1]
    conv = dis[:, :, None] * (aggsum + hp1_ref[...]) + b1_ref[...][None]
    x1 = jax.nn.relu(conv) + x_ref[...][None]
    x1_ref[...] = x1
    q = jnp.dot(x1, w2_ref[...], preferred_element_type=jnp.float32,
                precision=lax.Precision.HIGHEST)
    hp2_ref[...] = dis[:, :, None] * q


def _k_post_body(agg_ref, hp2_ref, x1_ref, b2_ref, deg_ref, mask_ref,
                 wc1_ref, bc1_ref, wc2_ref, bc2_ref, out_ref):
    dis = _dis_from_deg(deg_ref[...])
    x2 = (dis[:, :, None] * (agg_ref[0] + agg_ref[1] + hp2_ref[...])
          + b2_ref[...][None] + x1_ref[...])
    a = jnp.mean(x2, axis=0) * mask_ref[...]
    h = jax.nn.relu(jnp.dot(a, wc1_ref[...], preferred_element_type=jnp.float32,
                            precision=lax.Precision.HIGHEST) + bc1_ref[...])
    logit = jnp.dot(h, wc2_ref[...], preferred_element_type=jnp.float32,
                    precision=lax.Precision.HIGHEST) + bc2_ref[...]
    out_ref[...] = jax.nn.sigmoid(logit)


def _full(shape):
    return pl.BlockSpec(shape, lambda i: (0,) * len(shape))


def _rows(shape, axis):
    def imap(i, axis=axis, rank=len(shape)):
        return tuple(i if d == axis else 0 for d in range(rank))
    return pl.BlockSpec(shape, imap)


def kernel(node_features, edge_index, post_mask, W1, b1, W2, b2,
           Wc1, bc1, Wc2, bc2):
    N, D = node_features.shape
    S = edge_index.shape[0]
    E = edge_index.shape[2]
    H = Wc1.shape[1]

    shift = (jnp.arange(S, dtype=jnp.int32) * N)[:, None]

    def _staged(idx, padval):
        a = idx.reshape(S, NW, _NCH, CHUNK)
        a = jnp.pad(a, ((0, 0), (0, 0), (0, _NCHP - _NCH), (0, 0)),
                    constant_values=padval)
        return a.reshape(S * NW * _NCHP, CHUNK)

    srcs_flat = _staged(edge_index[:, 0, :] + shift, 0)
    dsts_flat = _staged(edge_index[:, 1, :], _AGGROWS - 1)

    ones_rows = jnp.zeros((CHUNK, D), jnp.float32).at[:, 0].set(1.0)
    zeros_agg = jnp.zeros((_AGGTILE, D), jnp.float32)
    maskf = post_mask.astype(jnp.float32).reshape(N, 1)
    b1r = b1.reshape(1, D)
    b2r = b2.reshape(1, D)
    bc1r = bc1.reshape(1, H)

    degtab = _sc_deg(dsts_flat, ones_rows, zeros_agg)
    deg4 = degtab.reshape(NC, S, _AGGROWS, D)[:, :, :N, :1]

    nb = N // _R
    p, hp1 = pl.pallas_call(
        _k_pre_body,
        grid=(nb,),
        in_specs=[_rows((_R, D), 0), _full((D, D)),
                  _rows((NC, S, _R, 1), 2)],
        out_specs=[_rows((_R, D), 0), _rows((S, _R, D), 1)],
        out_shape=[jax.ShapeDtypeStruct((N, D), jnp.float32),
                   jax.ShapeDtypeStruct((S, N, D), jnp.float32)],
    )(node_features, W1, deg4)

    agg1 = _sc_agg(srcs_flat, dsts_flat, hp1.reshape(S * N, D), zeros_agg)
    agg1 = agg1.reshape(NC, S, _AGGROWS, D)[:, :, :N]

    x1, hp2 = pl.pallas_call(
        _k_mid_body,
        grid=(nb,),
        in_specs=[_rows((NC, S, _R, D), 2), _rows((S, _R, D), 1),
                  _rows((_R, D), 0), _full((1, D)),
                  _rows((NC, S, _R, 1), 2), _full((D, D))],
        out_specs=[_rows((S, _R, D), 1), _rows((S, _R, D), 1)],
        out_shape=[jax.ShapeDtypeStruct((S, N, D), jnp.float32),
                   jax.ShapeDtypeStruct((S, N, D), jnp.float32)],
    )(agg1, hp1, node_features, b1r, deg4, W2)

    agg2 = _sc_agg(srcs_flat, dsts_flat, hp2.reshape(S * N, D), zeros_agg)
    agg2 = agg2.reshape(NC, S, _AGGROWS, D)[:, :, :N]

    out = pl.pallas_call(
        _k_post_body,
        grid=(nb,),
        in_specs=[_rows((NC, S, _R, D), 2), _rows((S, _R, D), 1),
                  _rows((S, _R, D), 1), _full((1, D)),
                  _rows((NC, S, _R, 1), 2), _rows((_R, 1), 0),
                  _full((D, H)), _full((1, H)), _full((H, 1)), _full((1, 1))],
        out_specs=[_rows((_R, 1), 0)],
        out_shape=[jax.ShapeDtypeStruct((N, 1), jnp.float32)],
    )(agg2, hp2, x1, b2r, deg4, maskf, Wc1, bc1r, Wc2, bc2.reshape(1, 1))[0]
    return out.reshape(N)
